# baseline (speedup 1.0000x reference)
import functools

import jax
import jax.numpy as jnp
from jax import lax
from jax.experimental import pallas as pl
from jax.experimental.pallas import tpu as pltpu
from jax.experimental.pallas import tpu_sc as plsc

_N, _C, _H, _W = 4, 96, 224, 224
_HW = _H * _W
_PT = _N * _HW
_NC, _NS, _L = 2, 16, 16
_NW = _NC * _NS
_PPW = _PT // _NW
_CHUNK = 128
_NCHUNK = _PPW // _CHUNK

_mesh = plsc.VectorSubcoreMesh(core_axis_name="c", subcore_axis_name="s")


@functools.partial(
    pl.kernel,
    mesh=_mesh,
    out_type=jax.ShapeDtypeStruct((_PT, _C), jnp.float32),
    scratch_types=[
        pltpu.VMEM((2, 4, _CHUNK), jnp.int32),
        pltpu.VMEM((2, 4, _CHUNK), jnp.float32),
        pltpu.VMEM((_CHUNK,), jnp.float32),
        pltpu.VMEM((_CHUNK,), jnp.float32),
        pltpu.VMEM((2 * 4 * _CHUNK, _C), jnp.float32),
        pltpu.VMEM((_CHUNK, _C), jnp.float32),
        pltpu.SemaphoreType.DMA,
        pltpu.SemaphoreType.DMA,
    ],
    compiler_params=pltpu.CompilerParams(use_tc_tiling_on_sc=False),
)
def _sc_pass(table, gx, gy, out, idx_s, w_s, gxv, gyv, rows, outv, sem0, sem1):
    wid = lax.axis_index("s") * _NC + lax.axis_index("c")
    pix0 = wid * _PPW
    rowbase = (pix0 // _HW) * _HW
    sems = [sem0, sem1]

    def compute_idx(pix, p):
        pltpu.sync_copy(gx.at[pl.ds(pix, _CHUNK)], gxv)
        pltpu.sync_copy(gy.at[pl.ds(pix, _CHUNK)], gyv)
        for v in range(_CHUNK // _L):
            sl = pl.ds(v * _L, _L)
            x = gxv[sl]
            y = gyv[sl]
            fx = ((x + 1.0) * _W - 1.0) * 0.5
            fy = ((y + 1.0) * _H - 1.0) * 0.5
            tx = fx.astype(jnp.int32)
            ty = fy.astype(jnp.int32)
            ix0 = jnp.where(tx.astype(jnp.float32) > fx, tx - 1, tx)
            iy0 = jnp.where(ty.astype(jnp.float32) > fy, ty - 1, ty)
            wx1 = fx - ix0.astype(jnp.float32)
            wy1 = fy - iy0.astype(jnp.float32)
            wx0 = 1.0 - wx1
            wy0 = 1.0 - wy1
            ix1 = ix0 + 1
            iy1 = iy0 + 1
            vx0 = jnp.where((ix0 >= 0) & (ix0 <= _W - 1), 1.0, 0.0)
            vx1 = jnp.where((ix1 >= 0) & (ix1 <= _W - 1), 1.0, 0.0)
            vy0 = jnp.where((iy0 >= 0) & (iy0 <= _H - 1), 1.0, 0.0)
            vy1 = jnp.where((iy1 >= 0) & (iy1 <= _H - 1), 1.0, 0.0)
            cx0 = jnp.clip(ix0, 0, _W - 1)
            cx1 = jnp.clip(ix1, 0, _W - 1)
            cy0 = jnp.clip(iy0, 0, _H - 1) * _W
            cy1 = jnp.clip(iy1, 0, _H - 1) * _W
            idx_s[p, 0, sl] = rowbase + cy0 + cx0
            idx_s[p, 1, sl] = rowbase + cy0 + cx1
            idx_s[p, 2, sl] = rowbase + cy1 + cx0
            idx_s[p, 3, sl] = rowbase + cy1 + cx1
            w_s[p, 0, sl] = wy0 * wx0 * (vy0 * vx0)
            w_s[p, 1, sl] = wy0 * wx1 * (vy0 * vx1)
            w_s[p, 2, sl] = wy1 * wx0 * (vy1 * vx0)
            w_s[p, 3, sl] = wy1 * wx1 * (vy1 * vx1)

    def fire(p):
        for k in range(4):
            pltpu.async_copy(
                table.at[idx_s.at[p, k]],
                rows.at[pl.ds((p * 4 + k) * _CHUNK, _CHUNK)],
                sems[p],
            )

    def drain(p):
        for k in range(4):
            pltpu.make_async_copy(
                table.at[idx_s.at[p, k]],
                rows.at[pl.ds((p * 4 + k) * _CHUNK, _CHUNK)],
                sems[p],
            ).wait()

    def accum(pix, p):
        def grp_body(g, c2):
            wvs = [w_s[p, k, pl.ds(g * _L, _L)] for k in range(4)]
            for i in range(_L):
                q = g * _L + i
                wb = [jnp.full((_L,), wvs[k][i], jnp.float32) for k in range(4)]
                for j in range(_C // _L):
                    cs = pl.ds(j * _L, _L)
                    acc = wb[0] * rows[(p * 4 + 0) * _CHUNK + q, cs]
                    acc = acc + wb[1] * rows[(p * 4 + 1) * _CHUNK + q, cs]
                    acc = acc + wb[2] * rows[(p * 4 + 2) * _CHUNK + q, cs]
                    acc = acc + wb[3] * rows[(p * 4 + 3) * _CHUNK + q, cs]
                    outv[q, cs] = acc
            return c2

        lax.fori_loop(0, _CHUNK // _L, grp_body, 0)
        pltpu.sync_copy(outv, out.at[pl.ds(pix, _CHUNK)])

    compute_idx(pix0, 0)
    fire(0)

    def pair_body(g, carry):
        pix = pix0 + (2 * g) * _CHUNK
        for b in range(2):
            compute_idx(pix + (b + 1) * _CHUNK, 1 - b)
            fire(1 - b)
            drain(b)
            accum(pix + b * _CHUNK, b)
        return carry

    lax.fori_loop(0, (_NCHUNK - 1) // 2, pair_body, 0)
    drain(0)
    accum(pix0 + (_NCHUNK - 1) * _CHUNK, 0)


_TB = 512
_NB = _HW // _TB


def _t_in_body(x_ref, o_ref):
    o_ref[...] = jnp.transpose(x_ref[...], (0, 2, 1))


def _tc_nchw_to_nhwc(x):
    return pl.pallas_call(
        _t_in_body,
        grid=(_N, _NB),
        in_specs=[pl.BlockSpec((1, _C, _TB), lambda n, j: (n, 0, j))],
        out_specs=pl.BlockSpec((1, _TB, _C), lambda n, j: (n, j, 0)),
        out_shape=jax.ShapeDtypeStruct((_N, _HW, _C), jnp.float32),
    )(x)


def _tc_nhwc_to_nchw(x):
    return pl.pallas_call(
        _t_in_body,
        grid=(_N, _NB),
        in_specs=[pl.BlockSpec((1, _TB, _C), lambda n, j: (n, j, 0))],
        out_specs=pl.BlockSpec((1, _C, _TB), lambda n, j: (n, 0, j)),
        out_shape=jax.ShapeDtypeStruct((_N, _C, _HW), jnp.float32),
    )(x)


def kernel(feature, grid):
    featf = _tc_nchw_to_nhwc(feature.reshape(_N, _C, _HW)).reshape(_PT, _C)
    gx = grid[..., 0].reshape(_PT)
    gy = grid[..., 1].reshape(_PT)
    o1 = _sc_pass(featf, gx, gy)
    o2 = _sc_pass(o1, gx, gy)
    return _tc_nhwc_to_nchw(o2.reshape(_N, _HW, _C)).reshape(_N, _C, _H, _W)


# speedup vs baseline: 1.2494x; 1.2494x over previous
import functools

import jax
import jax.numpy as jnp
from jax import lax
from jax.experimental import pallas as pl
from jax.experimental.pallas import tpu as pltpu
from jax.experimental.pallas import tpu_sc as plsc

_N, _C, _H, _W = 4, 96, 224, 224
_HW = _H * _W
_PT = _N * _HW
_NC, _NS, _L = 2, 16, 16
_NW = _NC * _NS
_PPW = _PT // _NW
_CHUNK = 128
_NCHUNK = _PPW // _CHUNK

_mesh = plsc.VectorSubcoreMesh(core_axis_name="c", subcore_axis_name="s")


@functools.partial(
    pl.kernel,
    mesh=_mesh,
    out_type=jax.ShapeDtypeStruct((_PT, _C), jnp.float32),
    scratch_types=[
        pltpu.VMEM((2, 4, _CHUNK), jnp.int32),
        pltpu.VMEM((2, 4, _CHUNK), jnp.float32),
        pltpu.VMEM((_CHUNK,), jnp.float32),
        pltpu.VMEM((_CHUNK,), jnp.float32),
        pltpu.VMEM((2 * 4 * _CHUNK, _C), jnp.float32),
        pltpu.VMEM((_CHUNK, _C), jnp.float32),
        pltpu.SemaphoreType.DMA,
        pltpu.SemaphoreType.DMA,
    ],
    compiler_params=pltpu.CompilerParams(use_tc_tiling_on_sc=False),
)
def _sc_pass(table, gx, gy, out, idx_s, w_s, gxv, gyv, rows, outv, sem0, sem1):
    wid = lax.axis_index("s") * _NC + lax.axis_index("c")
    pix0 = wid * _PPW
    rowbase = (pix0 // _HW) * _HW
    sems = [sem0, sem1]

    def compute_idx(pix, p):
        pltpu.sync_copy(gx.at[pl.ds(pix, _CHUNK)], gxv)
        pltpu.sync_copy(gy.at[pl.ds(pix, _CHUNK)], gyv)
        for v in range(_CHUNK // _L):
            sl = pl.ds(v * _L, _L)
            x = gxv[sl]
            y = gyv[sl]
            fx = ((x + 1.0) * _W - 1.0) * 0.5
            fy = ((y + 1.0) * _H - 1.0) * 0.5
            tx = fx.astype(jnp.int32)
            ty = fy.astype(jnp.int32)
            ix0 = jnp.where(tx.astype(jnp.float32) > fx, tx - 1, tx)
            iy0 = jnp.where(ty.astype(jnp.float32) > fy, ty - 1, ty)
            wx1 = fx - ix0.astype(jnp.float32)
            wy1 = fy - iy0.astype(jnp.float32)
            wx0 = 1.0 - wx1
            wy0 = 1.0 - wy1
            ix1 = ix0 + 1
            iy1 = iy0 + 1
            vx0 = jnp.where((ix0 >= 0) & (ix0 <= _W - 1), 1.0, 0.0)
            vx1 = jnp.where((ix1 >= 0) & (ix1 <= _W - 1), 1.0, 0.0)
            vy0 = jnp.where((iy0 >= 0) & (iy0 <= _H - 1), 1.0, 0.0)
            vy1 = jnp.where((iy1 >= 0) & (iy1 <= _H - 1), 1.0, 0.0)
            cx0 = jnp.clip(ix0, 0, _W - 1)
            cx1 = jnp.clip(ix1, 0, _W - 1)
            cy0 = jnp.clip(iy0, 0, _H - 1) * _W
            cy1 = jnp.clip(iy1, 0, _H - 1) * _W
            idx_s[p, 0, sl] = rowbase + cy0 + cx0
            idx_s[p, 1, sl] = rowbase + cy0 + cx1
            idx_s[p, 2, sl] = rowbase + cy1 + cx0
            idx_s[p, 3, sl] = rowbase + cy1 + cx1
            w_s[p, 0, sl] = wy0 * wx0 * (vy0 * vx0)
            w_s[p, 1, sl] = wy0 * wx1 * (vy0 * vx1)
            w_s[p, 2, sl] = wy1 * wx0 * (vy1 * vx0)
            w_s[p, 3, sl] = wy1 * wx1 * (vy1 * vx1)

    def fire(p):
        for k in range(4):
            pltpu.async_copy(
                table.at[idx_s.at[p, k]],
                rows.at[pl.ds((p * 4 + k) * _CHUNK, _CHUNK)],
                sems[p],
            )

    def drain(p):
        for k in range(4):
            pltpu.make_async_copy(
                table.at[idx_s.at[p, k]],
                rows.at[pl.ds((p * 4 + k) * _CHUNK, _CHUNK)],
                sems[p],
            ).wait()

    def accum(pix, p):
        def grp_body(g, c2):
            wvs = [w_s[p, k, pl.ds(g * _L, _L)] for k in range(4)]
            for i in range(_L):
                q = g * _L + i
                wb = [jnp.full((_L,), wvs[k][i], jnp.float32) for k in range(4)]
                for j in range(_C // _L):
                    cs = pl.ds(j * _L, _L)
                    acc = wb[0] * rows[(p * 4 + 0) * _CHUNK + q, cs]
                    acc = acc + wb[1] * rows[(p * 4 + 1) * _CHUNK + q, cs]
                    acc = acc + wb[2] * rows[(p * 4 + 2) * _CHUNK + q, cs]
                    acc = acc + wb[3] * rows[(p * 4 + 3) * _CHUNK + q, cs]
                    outv[q, cs] = acc
            return c2

        lax.fori_loop(0, _CHUNK // _L, grp_body, 0)
        pltpu.sync_copy(outv, out.at[pl.ds(pix, _CHUNK)])

    compute_idx(pix0, 0)
    fire(0)

    def pair_body(g, carry):
        pix = pix0 + (2 * g) * _CHUNK
        for b in range(2):
            compute_idx(pix + (b + 1) * _CHUNK, 1 - b)
            fire(1 - b)
            drain(b)
            accum(pix + b * _CHUNK, b)
        return carry

    lax.fori_loop(0, (_NCHUNK - 1) // 2, pair_body, 0)
    drain(0)
    accum(pix0 + (_NCHUNK - 1) * _CHUNK, 0)


_TB = 1792
_NB = _HW // _TB


def _t_in_body(x_ref, o_ref):
    o_ref[...] = jnp.transpose(x_ref[...], (0, 2, 1))


def _tc_nchw_to_nhwc(x):
    return pl.pallas_call(
        _t_in_body,
        grid=(_N, _NB),
        in_specs=[pl.BlockSpec((1, _C, _TB), lambda n, j: (n, 0, j))],
        out_specs=pl.BlockSpec((1, _TB, _C), lambda n, j: (n, j, 0)),
        out_shape=jax.ShapeDtypeStruct((_N, _HW, _C), jnp.float32),
    )(x)


def _tc_nhwc_to_nchw(x):
    return pl.pallas_call(
        _t_in_body,
        grid=(_N, _NB),
        in_specs=[pl.BlockSpec((1, _TB, _C), lambda n, j: (n, j, 0))],
        out_specs=pl.BlockSpec((1, _C, _TB), lambda n, j: (n, 0, j)),
        out_shape=jax.ShapeDtypeStruct((_N, _C, _HW), jnp.float32),
    )(x)


def kernel(feature, grid):
    featf = _tc_nchw_to_nhwc(feature.reshape(_N, _C, _HW)).reshape(_PT, _C)
    gx = grid[..., 0].reshape(_PT)
    gy = grid[..., 1].reshape(_PT)
    o1 = _sc_pass(featf, gx, gy)
    o2 = _sc_pass(o1, gx, gy)
    return _tc_nhwc_to_nchw(o2.reshape(_N, _HW, _C)).reshape(_N, _C, _H, _W)


# speedup vs baseline: 1.3252x; 1.0606x over previous
import functools

import jax
import jax.numpy as jnp
from jax import lax
from jax.experimental import pallas as pl
from jax.experimental.pallas import tpu as pltpu
from jax.experimental.pallas import tpu_sc as plsc

_N, _C, _H, _W = 4, 96, 224, 224
_HW = _H * _W
_PT = _N * _HW
_NC, _NS, _L = 2, 16, 16
_NW = _NC * _NS
_PPW = _PT // _NW
_CHUNK = 128
_NCHUNK = _PPW // _CHUNK

_mesh = plsc.VectorSubcoreMesh(core_axis_name="c", subcore_axis_name="s")


@functools.partial(
    pl.kernel,
    mesh=_mesh,
    out_type=jax.ShapeDtypeStruct((_PT, _C), jnp.float32),
    scratch_types=[
        pltpu.VMEM((2, 4, _CHUNK), jnp.int32),
        pltpu.VMEM((2, 4, _CHUNK), jnp.float32),
        pltpu.VMEM((_CHUNK,), jnp.float32),
        pltpu.VMEM((_CHUNK,), jnp.float32),
        pltpu.VMEM((2 * 4 * _CHUNK, _C), jnp.float32),
        pltpu.VMEM((_CHUNK, _C), jnp.float32),
        pltpu.SemaphoreType.DMA,
        pltpu.SemaphoreType.DMA,
    ],
    compiler_params=pltpu.CompilerParams(use_tc_tiling_on_sc=False),
)
def _sc_pass(table, gx, gy, out, idx_s, w_s, gxv, gyv, rows, outv, sem0, sem1):
    wid = lax.axis_index("s") * _NC + lax.axis_index("c")
    pix0 = wid * _PPW
    rowbase = (pix0 // _HW) * _HW
    sems = [sem0, sem1]

    def compute_idx(pix, p):
        pltpu.sync_copy(gx.at[pl.ds(pix, _CHUNK)], gxv)
        pltpu.sync_copy(gy.at[pl.ds(pix, _CHUNK)], gyv)
        for v in range(_CHUNK // _L):
            sl = pl.ds(v * _L, _L)
            x = gxv[sl]
            y = gyv[sl]
            fx = ((x + 1.0) * _W - 1.0) * 0.5
            fy = ((y + 1.0) * _H - 1.0) * 0.5
            tx = fx.astype(jnp.int32)
            ty = fy.astype(jnp.int32)
            ix0 = jnp.where(tx.astype(jnp.float32) > fx, tx - 1, tx)
            iy0 = jnp.where(ty.astype(jnp.float32) > fy, ty - 1, ty)
            wx1 = fx - ix0.astype(jnp.float32)
            wy1 = fy - iy0.astype(jnp.float32)
            wx0 = 1.0 - wx1
            wy0 = 1.0 - wy1
            ix1 = ix0 + 1
            iy1 = iy0 + 1
            vx0 = jnp.where((ix0 >= 0) & (ix0 <= _W - 1), 1.0, 0.0)
            vx1 = jnp.where((ix1 >= 0) & (ix1 <= _W - 1), 1.0, 0.0)
            vy0 = jnp.where((iy0 >= 0) & (iy0 <= _H - 1), 1.0, 0.0)
            vy1 = jnp.where((iy1 >= 0) & (iy1 <= _H - 1), 1.0, 0.0)
            cx0 = jnp.clip(ix0, 0, _W - 1)
            cx1 = jnp.clip(ix1, 0, _W - 1)
            cy0 = jnp.clip(iy0, 0, _H - 1) * _W
            cy1 = jnp.clip(iy1, 0, _H - 1) * _W
            idx_s[p, 0, sl] = rowbase + cy0 + cx0
            idx_s[p, 1, sl] = rowbase + cy0 + cx1
            idx_s[p, 2, sl] = rowbase + cy1 + cx0
            idx_s[p, 3, sl] = rowbase + cy1 + cx1
            w_s[p, 0, sl] = wy0 * wx0 * (vy0 * vx0)
            w_s[p, 1, sl] = wy0 * wx1 * (vy0 * vx1)
            w_s[p, 2, sl] = wy1 * wx0 * (vy1 * vx0)
            w_s[p, 3, sl] = wy1 * wx1 * (vy1 * vx1)

    def fire(p):
        for k in range(4):
            pltpu.async_copy(
                table.at[idx_s.at[p, k]],
                rows.at[pl.ds((p * 4 + k) * _CHUNK, _CHUNK)],
                sems[p],
            )

    def drain(p):
        for k in range(4):
            pltpu.make_async_copy(
                table.at[idx_s.at[p, k]],
                rows.at[pl.ds((p * 4 + k) * _CHUNK, _CHUNK)],
                sems[p],
            ).wait()

    def accum(pix, p):
        def grp_body(g, c2):
            wvs = [w_s[p, k, pl.ds(g * _L, _L)] for k in range(4)]
            for i in range(_L):
                q = g * _L + i
                wb = [jnp.full((_L,), wvs[k][i], jnp.float32) for k in range(4)]
                for j in range(_C // _L):
                    cs = pl.ds(j * _L, _L)
                    acc = wb[0] * rows[(p * 4 + 0) * _CHUNK + q, cs]
                    acc = acc + wb[1] * rows[(p * 4 + 1) * _CHUNK + q, cs]
                    acc = acc + wb[2] * rows[(p * 4 + 2) * _CHUNK + q, cs]
                    acc = acc + wb[3] * rows[(p * 4 + 3) * _CHUNK + q, cs]
                    outv[q, cs] = acc
            return c2

        lax.fori_loop(0, _CHUNK // _L, grp_body, 0)
        pltpu.sync_copy(outv, out.at[pl.ds(pix, _CHUNK)])

    compute_idx(pix0, 0)
    fire(0)

    def pair_body(g, carry):
        pix = pix0 + (2 * g) * _CHUNK
        for b in range(2):
            compute_idx(pix + (b + 1) * _CHUNK, 1 - b)
            fire(1 - b)
            drain(b)
            accum(pix + b * _CHUNK, b)
        return carry

    lax.fori_loop(0, (_NCHUNK - 1) // 2, pair_body, 0)
    drain(0)
    accum(pix0 + (_NCHUNK - 1) * _CHUNK, 0)


_TB = 3584
_NB = _HW // _TB


def _t_in_body(x_ref, o_ref):
    o_ref[...] = jnp.transpose(x_ref[...], (0, 2, 1))


def _tc_nchw_to_nhwc(x):
    return pl.pallas_call(
        _t_in_body,
        grid=(_N, _NB),
        in_specs=[pl.BlockSpec((1, _C, _TB), lambda n, j: (n, 0, j))],
        out_specs=pl.BlockSpec((1, _TB, _C), lambda n, j: (n, j, 0)),
        out_shape=jax.ShapeDtypeStruct((_N, _HW, _C), jnp.float32),
    )(x)


def _tc_nhwc_to_nchw(x):
    return pl.pallas_call(
        _t_in_body,
        grid=(_N, _NB),
        in_specs=[pl.BlockSpec((1, _TB, _C), lambda n, j: (n, j, 0))],
        out_specs=pl.BlockSpec((1, _C, _TB), lambda n, j: (n, 0, j)),
        out_shape=jax.ShapeDtypeStruct((_N, _C, _HW), jnp.float32),
    )(x)


def kernel(feature, grid):
    featf = _tc_nchw_to_nhwc(feature.reshape(_N, _C, _HW)).reshape(_PT, _C)
    gx = grid[..., 0].reshape(_PT)
    gy = grid[..., 1].reshape(_PT)
    o1 = _sc_pass(featf, gx, gy)
    o2 = _sc_pass(o1, gx, gy)
    return _tc_nhwc_to_nchw(o2.reshape(_N, _HW, _C)).reshape(_N, _C, _H, _W)
